# Initial kernel scaffold; baseline (speedup 1.0000x reference)
import jax
import jax.numpy as jnp
from jax.experimental import pallas as pl


def kernel(x, edge_index, W_in, b_in, W_h0, b_h0, W_h1, b_h1, W_out, b_out):
    raise NotImplementedError("write your pallas kernel here")


# speedup vs baseline: 16.6176x; 16.6176x over previous
import functools

import jax
import jax.numpy as jnp
from jax import lax
from jax.experimental import pallas as pl
from jax.experimental.pallas import tpu as pltpu
from jax.experimental.pallas import tpu_sc as plsc

N = 50000
E = 800000
HID = 64
HALF = 32

NC = 2
NS = 16

RPT = 3200
NPAD = RPT * NS
CH = 200
NCH = RPT // CH

EPT = E // NS
EB = 400
NB = EPT // EB
EPW = E // (NC * NS)

_mesh = plsc.VectorSubcoreMesh(
    core_axis_name="c", subcore_axis_name="s", num_cores=NC, num_subcores=NS
)

@functools.partial(
    pl.kernel,
    out_type=jax.ShapeDtypeStruct((NC * NPAD,), jnp.float32),
    mesh=_mesh,
    scratch_types=[
        pltpu.VMEM((EPW,), jnp.int32),
        pltpu.VMEM((EPW,), jnp.float32),
        pltpu.VMEM((RPT,), jnp.float32),
        pltpu.VMEM_SHARED((NPAD,), jnp.float32),
    ],
)
def _sc_degree(dst_hbm, ones_hbm, deg_hbm, dst_v, ones_v, stg_v, acc_sh):
    c = lax.axis_index("c")
    s = lax.axis_index("s")
    row = s * RPT

    def zero_body(i, carry):
        stg_v[pl.ds(i * 16, 16)] = jnp.zeros((16,), jnp.float32)
        return carry

    lax.fori_loop(0, RPT // 16, zero_body, 0)
    pltpu.sync_copy(stg_v, acc_sh.at[pl.ds(row, RPT)])
    plsc.subcore_barrier()

    wid = c * NS + s
    pltpu.sync_copy(dst_hbm.at[pl.ds(wid * EPW, EPW)], dst_v)
    pltpu.sync_copy(ones_hbm, ones_v)
    pltpu.sync_copy(ones_v, acc_sh.at[dst_v], add=True)
    plsc.subcore_barrier()

    pltpu.sync_copy(acc_sh.at[pl.ds(row, RPT)], stg_v)
    pltpu.sync_copy(stg_v, deg_hbm.at[pl.ds(c * NPAD + row, RPT)])


@functools.partial(
    pl.kernel,
    out_type=jax.ShapeDtypeStruct((NC, NPAD, HALF), jnp.float32),
    mesh=_mesh,
    scratch_types=[
        pltpu.VMEM((EB,), jnp.int32),
        pltpu.VMEM((EB,), jnp.int32),
        pltpu.VMEM((EB, HALF), jnp.float32),
        pltpu.VMEM((CH, HALF), jnp.float32),
        pltpu.VMEM_SHARED((NPAD, HALF), jnp.float32),
        pltpu.SemaphoreType.DMA,
    ],
    compiler_params=pltpu.CompilerParams(use_tc_tiling_on_sc=False),
)
def _sc_aggregate(src_hbm, dst_hbm, g0_hbm, g1_hbm, out_hbm,
                  src_v, dst_v, rows_v, stg_v, acc_sh, sem):
    c = lax.axis_index("c")
    s = lax.axis_index("s")
    row = s * RPT

    def zero_body(i, carry):
        stg_v[i, pl.ds(0, 16)] = jnp.zeros((16,), jnp.float32)
        stg_v[i, pl.ds(16, 16)] = jnp.zeros((16,), jnp.float32)
        return carry

    lax.fori_loop(0, CH, zero_body, 0)

    def zero_chunk(k, carry):
        pltpu.sync_copy(stg_v, acc_sh.at[pl.ds(row + k * CH, CH)])
        return carry

    lax.fori_loop(0, NCH, zero_chunk, 0)
    plsc.subcore_barrier()

    def body(i, carry):
        base = s * EPT + i * EB
        pltpu.sync_copy(src_hbm.at[pl.ds(base, EB)], src_v)
        pltpu.sync_copy(dst_hbm.at[pl.ds(base, EB)], dst_v)

        @pl.when(c == 0)
        def _():
            pltpu.async_copy(g0_hbm.at[src_v], rows_v, sem).wait()

        @pl.when(c == 1)
        def _():
            pltpu.async_copy(g1_hbm.at[src_v], rows_v, sem).wait()

        pltpu.sync_copy(rows_v, acc_sh.at[dst_v], add=True)
        return carry

    lax.fori_loop(0, NB, body, 0)
    plsc.subcore_barrier()

    def drain_chunk(k, carry):
        pltpu.sync_copy(acc_sh.at[pl.ds(row + k * CH, CH)], stg_v)
        pltpu.sync_copy(stg_v, out_hbm.at[c, pl.ds(row + k * CH, CH)])
        return carry

    lax.fori_loop(0, NCH, drain_chunk, 0)


BLK = 2000
GRID = N // BLK


def _dis(d0_blk, d1_blk):
    return lax.rsqrt(d0_blk + d1_blk + 1.0)


def _tc_first_body(x_ref, w_ref, d0_ref, d1_ref, g0_ref, g1_ref):
    dis = _dis(d0_ref[...], d1_ref[...])
    g = dis * jnp.dot(x_ref[...], w_ref[...], preferred_element_type=jnp.float32)
    g0_ref[...] = g[:, :HALF]
    g1_ref[...] = g[:, HALF:]


_tc_first = pl.pallas_call(
    _tc_first_body,
    grid=(GRID,),
    in_specs=[
        pl.BlockSpec((BLK, 4), lambda i: (i, 0)),
        pl.BlockSpec((4, HID), lambda i: (0, 0)),
        pl.BlockSpec((BLK, 1), lambda i: (i, 0)),
        pl.BlockSpec((BLK, 1), lambda i: (i, 0)),
    ],
    out_specs=[
        pl.BlockSpec((BLK, HALF), lambda i: (i, 0)),
        pl.BlockSpec((BLK, HALF), lambda i: (i, 0)),
    ],
    out_shape=[jax.ShapeDtypeStruct((N, HALF), jnp.float32)] * 2,
)


def _tc_mid_body(s0_ref, s1_ref, g0_ref, g1_ref, w_ref, b_ref, d0_ref, d1_ref,
                 o0_ref, o1_ref):
    dis = _dis(d0_ref[...], d1_ref[...])
    ssum0 = s0_ref[...] + g0_ref[...]
    ssum1 = s1_ref[...] + g1_ref[...]
    pre = dis * jnp.concatenate([ssum0, ssum1], axis=1) + b_ref[...]
    a = jnp.maximum(pre, 0.0)
    g = dis * jnp.dot(a, w_ref[...], preferred_element_type=jnp.float32)
    o0_ref[...] = g[:, :HALF]
    o1_ref[...] = g[:, HALF:]


_tc_mid = pl.pallas_call(
    _tc_mid_body,
    grid=(GRID,),
    in_specs=[
        pl.BlockSpec((BLK, HALF), lambda i: (i, 0)),
        pl.BlockSpec((BLK, HALF), lambda i: (i, 0)),
        pl.BlockSpec((BLK, HALF), lambda i: (i, 0)),
        pl.BlockSpec((BLK, HALF), lambda i: (i, 0)),
        pl.BlockSpec((HID, HID), lambda i: (0, 0)),
        pl.BlockSpec((1, HID), lambda i: (0, 0)),
        pl.BlockSpec((BLK, 1), lambda i: (i, 0)),
        pl.BlockSpec((BLK, 1), lambda i: (i, 0)),
    ],
    out_specs=[
        pl.BlockSpec((BLK, HALF), lambda i: (i, 0)),
        pl.BlockSpec((BLK, HALF), lambda i: (i, 0)),
    ],
    out_shape=[jax.ShapeDtypeStruct((N, HALF), jnp.float32)] * 2,
)


def _tc_last_body(s0_ref, s1_ref, g0_ref, g1_ref, w_ref, b_ref, d0_ref, d1_ref,
                  o_ref):
    dis = _dis(d0_ref[...], d1_ref[...])
    ssum0 = s0_ref[...] + g0_ref[...]
    ssum1 = s1_ref[...] + g1_ref[...]
    pre = dis * jnp.concatenate([ssum0, ssum1], axis=1)
    o_ref[...] = jnp.dot(pre, w_ref[...], preferred_element_type=jnp.float32) + b_ref[...]


_tc_last = pl.pallas_call(
    _tc_last_body,
    grid=(GRID,),
    in_specs=[
        pl.BlockSpec((BLK, HALF), lambda i: (i, 0)),
        pl.BlockSpec((BLK, HALF), lambda i: (i, 0)),
        pl.BlockSpec((BLK, HALF), lambda i: (i, 0)),
        pl.BlockSpec((BLK, HALF), lambda i: (i, 0)),
        pl.BlockSpec((HID, 1), lambda i: (0, 0)),
        pl.BlockSpec((1, 1), lambda i: (0, 0)),
        pl.BlockSpec((BLK, 1), lambda i: (i, 0)),
        pl.BlockSpec((BLK, 1), lambda i: (i, 0)),
    ],
    out_specs=pl.BlockSpec((BLK, 1), lambda i: (i, 0)),
    out_shape=jax.ShapeDtypeStruct((N, 1), jnp.float32),
)


def kernel(x, edge_index, W_in, b_in, W_h0, b_h0, W_h1, b_h1, W_out, b_out):
    src = edge_index[0]
    dst = edge_index[1]

    ones_deg = jnp.ones((EPW,), jnp.float32)

    deg = _sc_degree(dst, ones_deg)
    d0 = deg[:N].reshape(N, 1)
    d1 = deg[NPAD:NPAD + N].reshape(N, 1)

    def agg(g0, g1):
        S = _sc_aggregate(src, dst, g0, g1)
        return S[0, :N], S[1, :N]

    b_in2 = b_in.reshape(1, HID)
    b_h02 = b_h0.reshape(1, HID)
    b_h12 = b_h1.reshape(1, HID)
    eye = jnp.eye(HID, dtype=jnp.float32)

    g0, g1 = _tc_first(x, W_in, d0, d1)
    s0, s1 = agg(g0, g1)
    g0, g1 = _tc_mid(s0, s1, g0, g1, W_h0, b_in2, d0, d1)
    s0, s1 = agg(g0, g1)
    g0, g1 = _tc_mid(s0, s1, g0, g1, W_h1, b_h02, d0, d1)
    s0, s1 = agg(g0, g1)
    g0, g1 = _tc_mid(s0, s1, g0, g1, eye, b_h12, d0, d1)
    s0, s1 = agg(g0, g1)
    return _tc_last(s0, s1, g0, g1, W_out, b_out.reshape(1, 1), d0, d1)


# speedup vs baseline: 19.5026x; 1.1736x over previous
import functools

import jax
import jax.numpy as jnp
from jax import lax
from jax.experimental import pallas as pl
from jax.experimental.pallas import tpu as pltpu
from jax.experimental.pallas import tpu_sc as plsc

N = 50000
E = 800000
HID = 64
HALF = 32

NC = 2
NS = 16

RPT = 3200
NPAD = RPT * NS
CH = 200
NCH = RPT // CH

EPT = E // NS
EB = 200
NB = EPT // EB
NBUF = 2
EPW = E // (NC * NS)

_mesh = plsc.VectorSubcoreMesh(
    core_axis_name="c", subcore_axis_name="s", num_cores=NC, num_subcores=NS
)

@functools.partial(
    pl.kernel,
    out_type=jax.ShapeDtypeStruct((NC * NPAD,), jnp.float32),
    mesh=_mesh,
    scratch_types=[
        pltpu.VMEM((EPW,), jnp.int32),
        pltpu.VMEM((EPW,), jnp.float32),
        pltpu.VMEM((RPT,), jnp.float32),
        pltpu.VMEM_SHARED((NPAD,), jnp.float32),
    ],
)
def _sc_degree(dst_hbm, ones_hbm, deg_hbm, dst_v, ones_v, stg_v, acc_sh):
    c = lax.axis_index("c")
    s = lax.axis_index("s")
    row = s * RPT

    def zero_body(i, carry):
        stg_v[pl.ds(i * 16, 16)] = jnp.zeros((16,), jnp.float32)
        return carry

    lax.fori_loop(0, RPT // 16, zero_body, 0)
    pltpu.sync_copy(stg_v, acc_sh.at[pl.ds(row, RPT)])
    plsc.subcore_barrier()

    wid = c * NS + s
    pltpu.sync_copy(dst_hbm.at[pl.ds(wid * EPW, EPW)], dst_v)
    pltpu.sync_copy(ones_hbm, ones_v)
    pltpu.sync_copy(ones_v, acc_sh.at[dst_v], add=True)
    plsc.subcore_barrier()

    pltpu.sync_copy(acc_sh.at[pl.ds(row, RPT)], stg_v)
    pltpu.sync_copy(stg_v, deg_hbm.at[pl.ds(c * NPAD + row, RPT)])


@functools.partial(
    pl.kernel,
    out_type=jax.ShapeDtypeStruct((NC, NPAD, HALF), jnp.float32),
    mesh=_mesh,
    scratch_types=[
        [pltpu.VMEM((EB,), jnp.int32)] * NBUF,
        [pltpu.VMEM((EB,), jnp.int32)] * NBUF,
        [pltpu.VMEM((EB, HALF), jnp.float32)] * NBUF,
        pltpu.VMEM((CH, HALF), jnp.float32),
        pltpu.VMEM_SHARED((NPAD, HALF), jnp.float32),
        [pltpu.SemaphoreType.DMA] * NBUF,
        [pltpu.SemaphoreType.DMA] * NBUF,
        [pltpu.SemaphoreType.DMA] * NBUF,
    ],
    compiler_params=pltpu.CompilerParams(use_tc_tiling_on_sc=False),
)
def _sc_aggregate(src_hbm, dst_hbm, g0_hbm, g1_hbm, out_hbm,
                  src_v, dst_v, rows_v, stg_v, acc_sh, isem, gsem, ssem):
    c = lax.axis_index("c")
    s = lax.axis_index("s")
    row = s * RPT

    def zero_body(i, carry):
        stg_v[i, pl.ds(0, 16)] = jnp.zeros((16,), jnp.float32)
        stg_v[i, pl.ds(16, 16)] = jnp.zeros((16,), jnp.float32)
        return carry

    lax.fori_loop(0, CH, zero_body, 0)

    def zero_chunk(k, carry):
        pltpu.sync_copy(stg_v, acc_sh.at[pl.ds(row + k * CH, CH)])
        return carry

    lax.fori_loop(0, NCH, zero_chunk, 0)
    plsc.subcore_barrier()

    def outer(o, carry):
        for b in range(NBUF):
            i = o * NBUF + b
            base = s * EPT + i * EB

            @pl.when(o > 0)
            def _():
                pltpu.make_async_copy(
                    rows_v[b], acc_sh.at[dst_v[b]], ssem[b]).wait()

            pltpu.async_copy(src_hbm.at[pl.ds(base, EB)], src_v[b], isem[b])
            pltpu.async_copy(dst_hbm.at[pl.ds(base, EB)], dst_v[b], isem[b])
        for b in range(NBUF):
            i = o * NBUF + b
            base = s * EPT + i * EB
            pltpu.make_async_copy(
                src_hbm.at[pl.ds(base, EB)], src_v[b], isem[b]).wait()
            pltpu.make_async_copy(
                dst_hbm.at[pl.ds(base, EB)], dst_v[b], isem[b]).wait()

            @pl.when(c == 0)
            def _():
                pltpu.async_copy(g0_hbm.at[src_v[b]], rows_v[b], gsem[b])

            @pl.when(c == 1)
            def _():
                pltpu.async_copy(g1_hbm.at[src_v[b]], rows_v[b], gsem[b])
        for b in range(NBUF):
            pltpu.make_async_copy(
                g0_hbm.at[src_v[b]], rows_v[b], gsem[b]).wait()
            pltpu.async_copy(rows_v[b], acc_sh.at[dst_v[b]], ssem[b], add=True)
        return carry

    lax.fori_loop(0, NB // NBUF, outer, 0)
    for b in range(NBUF):
        pltpu.make_async_copy(rows_v[b], acc_sh.at[dst_v[b]], ssem[b]).wait()
    plsc.subcore_barrier()

    def drain_chunk(k, carry):
        pltpu.sync_copy(acc_sh.at[pl.ds(row + k * CH, CH)], stg_v)
        pltpu.sync_copy(stg_v, out_hbm.at[c, pl.ds(row + k * CH, CH)])
        return carry

    lax.fori_loop(0, NCH, drain_chunk, 0)


BLK = 2000
GRID = N // BLK


def _dis(d0_blk, d1_blk):
    return lax.rsqrt(d0_blk + d1_blk + 1.0)


def _tc_first_body(x_ref, w_ref, d0_ref, d1_ref, g0_ref, g1_ref):
    dis = _dis(d0_ref[...], d1_ref[...])
    g = dis * jnp.dot(x_ref[...], w_ref[...], preferred_element_type=jnp.float32)
    g0_ref[...] = g[:, :HALF]
    g1_ref[...] = g[:, HALF:]


_tc_first = pl.pallas_call(
    _tc_first_body,
    grid=(GRID,),
    in_specs=[
        pl.BlockSpec((BLK, 4), lambda i: (i, 0)),
        pl.BlockSpec((4, HID), lambda i: (0, 0)),
        pl.BlockSpec((BLK, 1), lambda i: (i, 0)),
        pl.BlockSpec((BLK, 1), lambda i: (i, 0)),
    ],
    out_specs=[
        pl.BlockSpec((BLK, HALF), lambda i: (i, 0)),
        pl.BlockSpec((BLK, HALF), lambda i: (i, 0)),
    ],
    out_shape=[jax.ShapeDtypeStruct((N, HALF), jnp.float32)] * 2,
)


def _tc_mid_body(s0_ref, s1_ref, g0_ref, g1_ref, w_ref, b_ref, d0_ref, d1_ref,
                 o0_ref, o1_ref):
    dis = _dis(d0_ref[...], d1_ref[...])
    ssum0 = s0_ref[...] + g0_ref[...]
    ssum1 = s1_ref[...] + g1_ref[...]
    pre = dis * jnp.concatenate([ssum0, ssum1], axis=1) + b_ref[...]
    a = jnp.maximum(pre, 0.0)
    g = dis * jnp.dot(a, w_ref[...], preferred_element_type=jnp.float32)
    o0_ref[...] = g[:, :HALF]
    o1_ref[...] = g[:, HALF:]


_tc_mid = pl.pallas_call(
    _tc_mid_body,
    grid=(GRID,),
    in_specs=[
        pl.BlockSpec((BLK, HALF), lambda i: (i, 0)),
        pl.BlockSpec((BLK, HALF), lambda i: (i, 0)),
        pl.BlockSpec((BLK, HALF), lambda i: (i, 0)),
        pl.BlockSpec((BLK, HALF), lambda i: (i, 0)),
        pl.BlockSpec((HID, HID), lambda i: (0, 0)),
        pl.BlockSpec((1, HID), lambda i: (0, 0)),
        pl.BlockSpec((BLK, 1), lambda i: (i, 0)),
        pl.BlockSpec((BLK, 1), lambda i: (i, 0)),
    ],
    out_specs=[
        pl.BlockSpec((BLK, HALF), lambda i: (i, 0)),
        pl.BlockSpec((BLK, HALF), lambda i: (i, 0)),
    ],
    out_shape=[jax.ShapeDtypeStruct((N, HALF), jnp.float32)] * 2,
)


def _tc_last_body(s0_ref, s1_ref, g0_ref, g1_ref, w_ref, b_ref, d0_ref, d1_ref,
                  o_ref):
    dis = _dis(d0_ref[...], d1_ref[...])
    ssum0 = s0_ref[...] + g0_ref[...]
    ssum1 = s1_ref[...] + g1_ref[...]
    pre = dis * jnp.concatenate([ssum0, ssum1], axis=1)
    o_ref[...] = jnp.dot(pre, w_ref[...], preferred_element_type=jnp.float32) + b_ref[...]


_tc_last = pl.pallas_call(
    _tc_last_body,
    grid=(GRID,),
    in_specs=[
        pl.BlockSpec((BLK, HALF), lambda i: (i, 0)),
        pl.BlockSpec((BLK, HALF), lambda i: (i, 0)),
        pl.BlockSpec((BLK, HALF), lambda i: (i, 0)),
        pl.BlockSpec((BLK, HALF), lambda i: (i, 0)),
        pl.BlockSpec((HID, 1), lambda i: (0, 0)),
        pl.BlockSpec((1, 1), lambda i: (0, 0)),
        pl.BlockSpec((BLK, 1), lambda i: (i, 0)),
        pl.BlockSpec((BLK, 1), lambda i: (i, 0)),
    ],
    out_specs=pl.BlockSpec((BLK, 1), lambda i: (i, 0)),
    out_shape=jax.ShapeDtypeStruct((N, 1), jnp.float32),
)


def kernel(x, edge_index, W_in, b_in, W_h0, b_h0, W_h1, b_h1, W_out, b_out):
    src = edge_index[0]
    dst = edge_index[1]

    ones_deg = jnp.ones((EPW,), jnp.float32)

    deg = _sc_degree(dst, ones_deg)
    d0 = deg[:N].reshape(N, 1)
    d1 = deg[NPAD:NPAD + N].reshape(N, 1)

    def agg(g0, g1):
        S = _sc_aggregate(src, dst, g0, g1)
        return S[0, :N], S[1, :N]

    b_in2 = b_in.reshape(1, HID)
    b_h02 = b_h0.reshape(1, HID)
    b_h12 = b_h1.reshape(1, HID)
    eye = jnp.eye(HID, dtype=jnp.float32)

    g0, g1 = _tc_first(x, W_in, d0, d1)
    s0, s1 = agg(g0, g1)
    g0, g1 = _tc_mid(s0, s1, g0, g1, W_h0, b_in2, d0, d1)
    s0, s1 = agg(g0, g1)
    g0, g1 = _tc_mid(s0, s1, g0, g1, W_h1, b_h02, d0, d1)
    s0, s1 = agg(g0, g1)
    g0, g1 = _tc_mid(s0, s1, g0, g1, eye, b_h12, d0, d1)
    s0, s1 = agg(g0, g1)
    return _tc_last(s0, s1, g0, g1, W_out, b_out.reshape(1, 1), d0, d1)


# speedup vs baseline: 28.9451x; 1.4842x over previous
import functools

import jax
import jax.numpy as jnp
from jax import lax
from jax.experimental import pallas as pl
from jax.experimental.pallas import tpu as pltpu
from jax.experimental.pallas import tpu_sc as plsc

N = 50000
E = 800000
HID = 64
HALF = 32

NC = 2
NS = 16

RPT = 3200
NPAD = RPT * NS
CH = 200
NCH = RPT // CH

EPT = E // NS
EB = 200
NB = EPT // EB
NBUF = 2

EPW = E // (NC * NS)
EBN = 1000
NBN = EPW // EBN
NBUFN = 5

_mesh = plsc.VectorSubcoreMesh(
    core_axis_name="c", subcore_axis_name="s", num_cores=NC, num_subcores=NS
)
_sc_params = pltpu.CompilerParams(use_tc_tiling_on_sc=False)


def _zero_fill(ref, nrows, width):
    per_row = width // 16
    if per_row == 0:
        def body(i, carry):
            ref[pl.ds(i * 16, 16)] = jnp.zeros((16,), jnp.float32)
            return carry
        lax.fori_loop(0, nrows * width // 16, body, 0)
    else:
        def body(i, carry):
            for k in range(per_row):
                ref[i, pl.ds(k * 16, 16)] = jnp.zeros((16,), jnp.float32)
            return carry
        lax.fori_loop(0, nrows, body, 0)


@functools.partial(
    pl.kernel,
    out_type=jax.ShapeDtypeStruct((NC * NPAD,), jnp.float32),
    mesh=_mesh,
    scratch_types=[
        pltpu.VMEM((EPW,), jnp.int32),
        pltpu.VMEM((EPW,), jnp.float32),
        pltpu.VMEM((RPT,), jnp.float32),
        pltpu.VMEM_SHARED((NPAD,), jnp.float32),
    ],
    compiler_params=_sc_params,
)
def _sc_degree(dst_hbm, ones_hbm, deg_hbm, dst_v, ones_v, stg_v, acc_sh):
    c = lax.axis_index("c")
    s = lax.axis_index("s")
    row = s * RPT
    _zero_fill(stg_v, RPT, 1)
    pltpu.sync_copy(stg_v, acc_sh.at[pl.ds(row, RPT)])
    plsc.subcore_barrier()

    wid = c * NS + s
    pltpu.sync_copy(dst_hbm.at[pl.ds(wid * EPW, EPW)], dst_v)
    pltpu.sync_copy(ones_hbm, ones_v)
    pltpu.sync_copy(ones_v, acc_sh.at[dst_v], add=True)
    plsc.subcore_barrier()

    pltpu.sync_copy(acc_sh.at[pl.ds(row, RPT)], stg_v)
    pltpu.sync_copy(stg_v, deg_hbm.at[pl.ds(c * NPAD + row, RPT)])


@functools.partial(
    pl.kernel,
    out_type=jax.ShapeDtypeStruct((NC, NPAD, HALF), jnp.float32),
    mesh=_mesh,
    scratch_types=[
        [pltpu.VMEM((EB,), jnp.int32)] * NBUF,
        [pltpu.VMEM((EB,), jnp.int32)] * NBUF,
        [pltpu.VMEM((EB, HALF), jnp.float32)] * NBUF,
        pltpu.VMEM((CH, HALF), jnp.float32),
        pltpu.VMEM_SHARED((NPAD, HALF), jnp.float32),
        [pltpu.SemaphoreType.DMA] * NBUF,
        [pltpu.SemaphoreType.DMA] * NBUF,
        [pltpu.SemaphoreType.DMA] * NBUF,
    ],
    compiler_params=_sc_params,
)
def _sc_agg_wide(src_hbm, dst_hbm, g0_hbm, g1_hbm, out_hbm,
                 src_v, dst_v, rows_v, stg_v, acc_sh, isem, gsem, ssem):
    c = lax.axis_index("c")
    s = lax.axis_index("s")
    row = s * RPT
    _zero_fill(stg_v, CH, HALF)

    def zero_chunk(k, carry):
        pltpu.sync_copy(stg_v, acc_sh.at[pl.ds(row + k * CH, CH)])
        return carry

    lax.fori_loop(0, NCH, zero_chunk, 0)
    plsc.subcore_barrier()

    def outer(o, carry):
        for b in range(NBUF):
            base = s * EPT + (o * NBUF + b) * EB

            @pl.when(o > 0)
            def _():
                pltpu.make_async_copy(
                    rows_v[b], acc_sh.at[dst_v[b]], ssem[b]).wait()

            pltpu.async_copy(src_hbm.at[pl.ds(base, EB)], src_v[b], isem[b])
            pltpu.async_copy(dst_hbm.at[pl.ds(base, EB)], dst_v[b], isem[b])
        for b in range(NBUF):
            base = s * EPT + (o * NBUF + b) * EB
            pltpu.make_async_copy(
                src_hbm.at[pl.ds(base, EB)], src_v[b], isem[b]).wait()
            pltpu.make_async_copy(
                dst_hbm.at[pl.ds(base, EB)], dst_v[b], isem[b]).wait()

            @pl.when(c == 0)
            def _():
                pltpu.async_copy(g0_hbm.at[src_v[b]], rows_v[b], gsem[b])

            @pl.when(c == 1)
            def _():
                pltpu.async_copy(g1_hbm.at[src_v[b]], rows_v[b], gsem[b])
        for b in range(NBUF):
            pltpu.make_async_copy(
                g0_hbm.at[src_v[b]], rows_v[b], gsem[b]).wait()
            pltpu.async_copy(rows_v[b], acc_sh.at[dst_v[b]], ssem[b], add=True)
        return carry

    lax.fori_loop(0, NB // NBUF, outer, 0)
    for b in range(NBUF):
        pltpu.make_async_copy(rows_v[b], acc_sh.at[dst_v[b]], ssem[b]).wait()
    plsc.subcore_barrier()

    def drain_chunk(k, carry):
        pltpu.sync_copy(acc_sh.at[pl.ds(row + k * CH, CH)], stg_v)
        pltpu.sync_copy(stg_v, out_hbm.at[c, pl.ds(row + k * CH, CH)])
        return carry

    lax.fori_loop(0, NCH, drain_chunk, 0)


def _make_sc_agg_narrow(width):
    out3 = width > 1
    out_shape = (NC, NPAD, width) if out3 else (NC * NPAD,)
    rows_shape = (EBN, width) if out3 else (EBN,)
    acc_shape = (NPAD, width) if out3 else (NPAD,)
    stg_shape = (RPT, width) if out3 else (RPT,)

    @functools.partial(
        pl.kernel,
        out_type=jax.ShapeDtypeStruct(out_shape, jnp.float32),
        mesh=_mesh,
        scratch_types=[
            [pltpu.VMEM((EBN,), jnp.int32)] * NBUFN,
            [pltpu.VMEM((EBN,), jnp.int32)] * NBUFN,
            [pltpu.VMEM(rows_shape, jnp.float32)] * NBUFN,
            pltpu.VMEM(stg_shape, jnp.float32),
            pltpu.VMEM_SHARED(acc_shape, jnp.float32),
            [pltpu.SemaphoreType.DMA] * NBUFN,
            [pltpu.SemaphoreType.DMA] * NBUFN,
            [pltpu.SemaphoreType.DMA] * NBUFN,
        ],
        compiler_params=_sc_params,
    )
    def agg(src_hbm, dst_hbm, g_hbm, zeros_hbm, out_hbm,
            src_v, dst_v, rows_v, stg_v, acc_sh, isem, gsem, ssem):
        c = lax.axis_index("c")
        s = lax.axis_index("s")
        row = s * RPT
        pltpu.sync_copy(zeros_hbm, stg_v)
        pltpu.sync_copy(stg_v, acc_sh.at[pl.ds(row, RPT)])
        plsc.subcore_barrier()

        def outer(o, carry):
            for b in range(NBUFN):
                base = (c * NS + s) * EPW + (o * NBUFN + b) * EBN

                @pl.when(o > 0)
                def _():
                    pltpu.make_async_copy(
                        rows_v[b], acc_sh.at[dst_v[b]], ssem[b]).wait()

                pltpu.async_copy(src_hbm.at[pl.ds(base, EBN)], src_v[b], isem[b])
                pltpu.async_copy(dst_hbm.at[pl.ds(base, EBN)], dst_v[b], isem[b])
            for b in range(NBUFN):
                base = (c * NS + s) * EPW + (o * NBUFN + b) * EBN
                pltpu.make_async_copy(
                    src_hbm.at[pl.ds(base, EBN)], src_v[b], isem[b]).wait()
                pltpu.make_async_copy(
                    dst_hbm.at[pl.ds(base, EBN)], dst_v[b], isem[b]).wait()
                pltpu.async_copy(g_hbm.at[src_v[b]], rows_v[b], gsem[b])
            for b in range(NBUFN):
                pltpu.make_async_copy(
                    g_hbm.at[src_v[b]], rows_v[b], gsem[b]).wait()
                pltpu.async_copy(rows_v[b], acc_sh.at[dst_v[b]], ssem[b],
                                 add=True)
            return carry

        lax.fori_loop(0, NBN // NBUFN, outer, 0)
        for b in range(NBUFN):
            pltpu.make_async_copy(rows_v[b], acc_sh.at[dst_v[b]], ssem[b]).wait()
        plsc.subcore_barrier()

        pltpu.sync_copy(acc_sh.at[pl.ds(row, RPT)], stg_v)
        if out3:
            pltpu.sync_copy(stg_v, out_hbm.at[c, pl.ds(row, RPT)])
        else:
            pltpu.sync_copy(stg_v, out_hbm.at[pl.ds(c * NPAD + row, RPT)])

    return agg


_sc_agg4 = _make_sc_agg_narrow(4)
_sc_agg1 = _make_sc_agg_narrow(1)


BLK = 2000
GRID = N // BLK


def _dis(d0_blk, d1_blk):
    return lax.rsqrt(d0_blk + d1_blk + 1.0)


def _node_spec(width):
    return pl.BlockSpec((BLK, width), lambda i: (i, 0))


def _pad_specs(width):
    return [
        pl.BlockSpec((1, BLK, width), lambda i: (0, i, 0)),
        pl.BlockSpec((1, BLK, width), lambda i: (1, i, 0)),
    ]


def _full_spec(shape):
    return pl.BlockSpec(shape, lambda i: tuple(0 for _ in shape))


def _tc_gx_body(x_ref, d0_ref, d1_ref, gx_ref):
    gx_ref[...] = _dis(d0_ref[...], d1_ref[...]) * x_ref[...]


_tc_gx = pl.pallas_call(
    _tc_gx_body,
    grid=(GRID,),
    in_specs=[_node_spec(4), _node_spec(1), _node_spec(1)],
    out_specs=_node_spec(4),
    out_shape=jax.ShapeDtypeStruct((N, 4), jnp.float32),
)


def _tc_l1_body(sx0_ref, sx1_ref, gx_ref, w_in_ref, w_h0_ref, b_ref,
                d0_ref, d1_ref, o0_ref, o1_ref):
    dis = _dis(d0_ref[...], d1_ref[...])
    pre = dis * (sx0_ref[0] + sx1_ref[0] + gx_ref[...])
    a = jnp.maximum(
        jnp.dot(pre, w_in_ref[...], preferred_element_type=jnp.float32)
        + b_ref[...], 0.0)
    g = dis * jnp.dot(a, w_h0_ref[...], preferred_element_type=jnp.float32)
    o0_ref[...] = g[:, :HALF]
    o1_ref[...] = g[:, HALF:]


_tc_l1 = pl.pallas_call(
    _tc_l1_body,
    grid=(GRID,),
    in_specs=_pad_specs(4) + [
        _node_spec(4), _full_spec((4, HID)), _full_spec((HID, HID)),
        _full_spec((1, HID)), _node_spec(1), _node_spec(1),
    ],
    out_specs=[_node_spec(HALF), _node_spec(HALF)],
    out_shape=[jax.ShapeDtypeStruct((N, HALF), jnp.float32)] * 2,
)


def _tc_mid_body(s0_ref, s1_ref, g0_ref, g1_ref, w_ref, b_ref,
                 d0_ref, d1_ref, o0_ref, o1_ref):
    dis = _dis(d0_ref[...], d1_ref[...])
    ssum0 = s0_ref[0] + g0_ref[...]
    ssum1 = s1_ref[0] + g1_ref[...]
    pre = dis * jnp.concatenate([ssum0, ssum1], axis=1) + b_ref[...]
    a = jnp.maximum(pre, 0.0)
    g = dis * jnp.dot(a, w_ref[...], preferred_element_type=jnp.float32)
    o0_ref[...] = g[:, :HALF]
    o1_ref[...] = g[:, HALF:]


_tc_mid = pl.pallas_call(
    _tc_mid_body,
    grid=(GRID,),
    in_specs=_pad_specs(HALF) + [
        _node_spec(HALF), _node_spec(HALF), _full_spec((HID, HID)),
        _full_spec((1, HID)), _node_spec(1), _node_spec(1),
    ],
    out_specs=[_node_spec(HALF), _node_spec(HALF)],
    out_shape=[jax.ShapeDtypeStruct((N, HALF), jnp.float32)] * 2,
)


def _tc_l4_body(s0_ref, s1_ref, g0_ref, g1_ref, w_ref, b_ref,
                d0_ref, d1_ref, o_ref):
    dis = _dis(d0_ref[...], d1_ref[...])
    ssum0 = s0_ref[0] + g0_ref[...]
    ssum1 = s1_ref[0] + g1_ref[...]
    pre = dis * jnp.concatenate([ssum0, ssum1], axis=1) + b_ref[...]
    a = jnp.maximum(pre, 0.0)
    o_ref[...] = dis * jnp.dot(a, w_ref[...], preferred_element_type=jnp.float32)


_tc_l4 = pl.pallas_call(
    _tc_l4_body,
    grid=(GRID,),
    in_specs=_pad_specs(HALF) + [
        _node_spec(HALF), _node_spec(HALF), _full_spec((HID, 1)),
        _full_spec((1, HID)), _node_spec(1), _node_spec(1),
    ],
    out_specs=_node_spec(1),
    out_shape=jax.ShapeDtypeStruct((N, 1), jnp.float32),
)


def _tc_out_body(s0_ref, s1_ref, g4_ref, b_ref, d0_ref, d1_ref, o_ref):
    dis = _dis(d0_ref[...], d1_ref[...])
    o_ref[...] = dis * (s0_ref[...] + s1_ref[...] + g4_ref[...]) + b_ref[...]


_tc_out = pl.pallas_call(
    _tc_out_body,
    grid=(GRID,),
    in_specs=[
        _node_spec(1), _node_spec(1), _node_spec(1), _full_spec((1, 1)),
        _node_spec(1), _node_spec(1),
    ],
    out_specs=_node_spec(1),
    out_shape=jax.ShapeDtypeStruct((N, 1), jnp.float32),
)


def kernel(x, edge_index, W_in, b_in, W_h0, b_h0, W_h1, b_h1, W_out, b_out):
    src = edge_index[0]
    dst = edge_index[1]

    ones_deg = jnp.ones((EPW,), jnp.float32)
    deg = _sc_degree(dst, ones_deg)
    d0 = deg[:N].reshape(N, 1)
    d1 = deg[NPAD:NPAD + N].reshape(N, 1)

    b_in2 = b_in.reshape(1, HID)
    b_h02 = b_h0.reshape(1, HID)
    b_h12 = b_h1.reshape(1, HID)

    zeros4 = jnp.zeros((RPT, 4), jnp.float32)
    zeros1 = jnp.zeros((RPT,), jnp.float32)

    gx = _tc_gx(x, d0, d1)
    Sx = _sc_agg4(src, dst, gx, zeros4)
    g0, g1 = _tc_l1(Sx, Sx, gx, W_in, W_h0, b_in2, d0, d1)
    S2 = _sc_agg_wide(src, dst, g0, g1)
    g0, g1 = _tc_mid(S2, S2, g0, g1, W_h1, b_h02, d0, d1)
    S3 = _sc_agg_wide(src, dst, g0, g1)
    g4 = _tc_l4(S3, S3, g0, g1, W_out, b_h12, d0, d1)
    S4 = _sc_agg1(src, dst, g4.reshape(N), zeros1)
    s40 = S4[:N].reshape(N, 1)
    s41 = S4[NPAD:NPAD + N].reshape(N, 1)
    return _tc_out(s40, s41, g4, b_out.reshape(1, 1), d0, d1)


# speedup vs baseline: 29.0394x; 1.0033x over previous
import functools

import jax
import jax.numpy as jnp
from jax import lax
from jax.experimental import pallas as pl
from jax.experimental.pallas import tpu as pltpu
from jax.experimental.pallas import tpu_sc as plsc

N = 50000
E = 800000
HID = 64
HALF = 32

NC = 2
NS = 16

RPT = 3200
NPAD = RPT * NS
CH = 200
NCH = RPT // CH

EPT = E // NS
EB = 200
NB = EPT // EB
NBUF = 2

EPW = E // (NC * NS)
W1 = 16
NBUFN = 5

_mesh = plsc.VectorSubcoreMesh(
    core_axis_name="c", subcore_axis_name="s", num_cores=NC, num_subcores=NS
)
_sc_params = pltpu.CompilerParams(use_tc_tiling_on_sc=False)


def _zero_fill(ref, nrows, width):
    per_row = width // 16
    if per_row == 0:
        def body(i, carry):
            ref[pl.ds(i * 16, 16)] = jnp.zeros((16,), jnp.float32)
            return carry
        lax.fori_loop(0, nrows * width // 16, body, 0)
    else:
        def body(i, carry):
            for k in range(per_row):
                ref[i, pl.ds(k * 16, 16)] = jnp.zeros((16,), jnp.float32)
            return carry
        lax.fori_loop(0, nrows, body, 0)


@functools.partial(
    pl.kernel,
    out_type=jax.ShapeDtypeStruct((NC * NPAD,), jnp.float32),
    mesh=_mesh,
    scratch_types=[
        pltpu.VMEM((EPW,), jnp.int32),
        pltpu.VMEM((EPW,), jnp.float32),
        pltpu.VMEM((RPT,), jnp.float32),
        pltpu.VMEM_SHARED((NPAD,), jnp.float32),
    ],
    compiler_params=_sc_params,
)
def _sc_degree(dst_hbm, ones_hbm, deg_hbm, dst_v, ones_v, stg_v, acc_sh):
    c = lax.axis_index("c")
    s = lax.axis_index("s")
    row = s * RPT
    _zero_fill(stg_v, RPT, 1)
    pltpu.sync_copy(stg_v, acc_sh.at[pl.ds(row, RPT)])
    plsc.subcore_barrier()

    wid = c * NS + s
    pltpu.sync_copy(dst_hbm.at[pl.ds(wid * EPW, EPW)], dst_v)
    pltpu.sync_copy(ones_hbm, ones_v)
    pltpu.sync_copy(ones_v, acc_sh.at[dst_v], add=True)
    plsc.subcore_barrier()

    pltpu.sync_copy(acc_sh.at[pl.ds(row, RPT)], stg_v)
    pltpu.sync_copy(stg_v, deg_hbm.at[pl.ds(c * NPAD + row, RPT)])


@functools.partial(
    pl.kernel,
    out_type=jax.ShapeDtypeStruct((NC, NPAD, HALF), jnp.float32),
    mesh=_mesh,
    scratch_types=[
        [pltpu.VMEM((EB,), jnp.int32)] * NBUF,
        [pltpu.VMEM((EB,), jnp.int32)] * NBUF,
        [pltpu.VMEM((EB, HALF), jnp.float32)] * NBUF,
        pltpu.VMEM((CH, HALF), jnp.float32),
        pltpu.VMEM_SHARED((NPAD, HALF), jnp.float32),
        [pltpu.SemaphoreType.DMA] * NBUF,
        [pltpu.SemaphoreType.DMA] * NBUF,
        [pltpu.SemaphoreType.DMA] * NBUF,
    ],
    compiler_params=_sc_params,
)
def _sc_agg_wide(src_hbm, dst_hbm, g0_hbm, g1_hbm, out_hbm,
                 src_v, dst_v, rows_v, stg_v, acc_sh, isem, gsem, ssem):
    c = lax.axis_index("c")
    s = lax.axis_index("s")
    row = s * RPT
    _zero_fill(stg_v, CH, HALF)

    def zero_chunk(k, carry):
        pltpu.sync_copy(stg_v, acc_sh.at[pl.ds(row + k * CH, CH)])
        return carry

    lax.fori_loop(0, NCH, zero_chunk, 0)
    plsc.subcore_barrier()

    def outer(o, carry):
        for b in range(NBUF):
            base = s * EPT + (o * NBUF + b) * EB

            @pl.when(o > 0)
            def _():
                pltpu.make_async_copy(
                    rows_v[b], acc_sh.at[dst_v[b]], ssem[b]).wait()

            pltpu.async_copy(src_hbm.at[pl.ds(base, EB)], src_v[b], isem[b])
            pltpu.async_copy(dst_hbm.at[pl.ds(base, EB)], dst_v[b], isem[b])
        for b in range(NBUF):
            base = s * EPT + (o * NBUF + b) * EB
            pltpu.make_async_copy(
                src_hbm.at[pl.ds(base, EB)], src_v[b], isem[b]).wait()
            pltpu.make_async_copy(
                dst_hbm.at[pl.ds(base, EB)], dst_v[b], isem[b]).wait()

            @pl.when(c == 0)
            def _():
                pltpu.async_copy(g0_hbm.at[src_v[b]], rows_v[b], gsem[b])

            @pl.when(c == 1)
            def _():
                pltpu.async_copy(g1_hbm.at[src_v[b]], rows_v[b], gsem[b])
        for b in range(NBUF):
            pltpu.make_async_copy(
                g0_hbm.at[src_v[b]], rows_v[b], gsem[b]).wait()
            pltpu.async_copy(rows_v[b], acc_sh.at[dst_v[b]], ssem[b], add=True)
        return carry

    lax.fori_loop(0, NB // NBUF, outer, 0)
    for b in range(NBUF):
        pltpu.make_async_copy(rows_v[b], acc_sh.at[dst_v[b]], ssem[b]).wait()
    plsc.subcore_barrier()

    def drain_chunk(k, carry):
        pltpu.sync_copy(acc_sh.at[pl.ds(row + k * CH, CH)], stg_v)
        pltpu.sync_copy(stg_v, out_hbm.at[c, pl.ds(row + k * CH, CH)])
        return carry

    lax.fori_loop(0, NCH, drain_chunk, 0)


def _make_sc_agg_narrow(width, ebn):
    out3 = width > 1
    nbn = EPW // ebn
    out_shape = (NC, NPAD, width) if out3 else (NC * NPAD,)
    rows_shape = (ebn, width) if out3 else (ebn,)
    acc_shape = (NPAD, width) if out3 else (NPAD,)
    stg_shape = (RPT, width) if out3 else (RPT,)

    @functools.partial(
        pl.kernel,
        out_type=jax.ShapeDtypeStruct(out_shape, jnp.float32),
        mesh=_mesh,
        scratch_types=[
            [pltpu.VMEM((ebn,), jnp.int32)] * NBUFN,
            [pltpu.VMEM((ebn,), jnp.int32)] * NBUFN,
            [pltpu.VMEM(rows_shape, jnp.float32)] * NBUFN,
            pltpu.VMEM(stg_shape, jnp.float32),
            pltpu.VMEM_SHARED(acc_shape, jnp.float32),
            [pltpu.SemaphoreType.DMA] * NBUFN,
            [pltpu.SemaphoreType.DMA] * NBUFN,
            [pltpu.SemaphoreType.DMA] * NBUFN,
        ],
        compiler_params=_sc_params,
    )
    def agg(src_hbm, dst_hbm, g_hbm, zeros_hbm, out_hbm,
            src_v, dst_v, rows_v, stg_v, acc_sh, isem, gsem, ssem):
        c = lax.axis_index("c")
        s = lax.axis_index("s")
        row = s * RPT
        pltpu.sync_copy(zeros_hbm, stg_v)
        pltpu.sync_copy(stg_v, acc_sh.at[pl.ds(row, RPT)])
        plsc.subcore_barrier()

        def outer(o, carry):
            for b in range(NBUFN):
                base = (c * NS + s) * EPW + (o * NBUFN + b) * ebn

                @pl.when(o > 0)
                def _():
                    pltpu.make_async_copy(
                        rows_v[b], acc_sh.at[dst_v[b]], ssem[b]).wait()

                pltpu.async_copy(src_hbm.at[pl.ds(base, ebn)], src_v[b], isem[b])
                pltpu.async_copy(dst_hbm.at[pl.ds(base, ebn)], dst_v[b], isem[b])
            for b in range(NBUFN):
                base = (c * NS + s) * EPW + (o * NBUFN + b) * ebn
                pltpu.make_async_copy(
                    src_hbm.at[pl.ds(base, ebn)], src_v[b], isem[b]).wait()
                pltpu.make_async_copy(
                    dst_hbm.at[pl.ds(base, ebn)], dst_v[b], isem[b]).wait()
                pltpu.async_copy(g_hbm.at[src_v[b]], rows_v[b], gsem[b])
            for b in range(NBUFN):
                pltpu.make_async_copy(
                    g_hbm.at[src_v[b]], rows_v[b], gsem[b]).wait()
                pltpu.async_copy(rows_v[b], acc_sh.at[dst_v[b]], ssem[b],
                                 add=True)
            return carry

        lax.fori_loop(0, nbn // NBUFN, outer, 0)
        for b in range(NBUFN):
            pltpu.make_async_copy(rows_v[b], acc_sh.at[dst_v[b]], ssem[b]).wait()
        plsc.subcore_barrier()

        pltpu.sync_copy(acc_sh.at[pl.ds(row, RPT)], stg_v)
        if out3:
            pltpu.sync_copy(stg_v, out_hbm.at[c, pl.ds(row, RPT)])
        else:
            pltpu.sync_copy(stg_v, out_hbm.at[pl.ds(c * NPAD + row, RPT)])

    return agg


_sc_agg16 = _make_sc_agg_narrow(W1, 200)
_sc_agg1 = _make_sc_agg_narrow(1, 1000)


BLK = 2000
GRID = N // BLK


def _dis(d0_blk, d1_blk):
    return lax.rsqrt(d0_blk + d1_blk + 1.0)


def _node_spec(width):
    return pl.BlockSpec((BLK, width), lambda i: (i, 0))


def _pad_specs(width):
    return [
        pl.BlockSpec((1, BLK, width), lambda i: (0, i, 0)),
        pl.BlockSpec((1, BLK, width), lambda i: (1, i, 0)),
    ]


def _full_spec(shape):
    return pl.BlockSpec(shape, lambda i: tuple(0 for _ in shape))


def _tc_gx_body(x_ref, d0_ref, d1_ref, gx_ref):
    gx = _dis(d0_ref[...], d1_ref[...]) * x_ref[...]
    gx_ref[...] = jnp.concatenate(
        [gx, jnp.zeros((BLK, W1 - 4), jnp.float32)], axis=1)


_tc_gx = pl.pallas_call(
    _tc_gx_body,
    grid=(GRID,),
    in_specs=[_node_spec(4), _node_spec(1), _node_spec(1)],
    out_specs=_node_spec(W1),
    out_shape=jax.ShapeDtypeStruct((N, W1), jnp.float32),
)


def _tc_l1_body(sx0_ref, sx1_ref, gx_ref, w_in_ref, w_h0_ref, b_ref,
                d0_ref, d1_ref, o0_ref, o1_ref):
    dis = _dis(d0_ref[...], d1_ref[...])
    pre = dis * (sx0_ref[0] + sx1_ref[0] + gx_ref[...])
    a = jnp.maximum(
        jnp.dot(pre, w_in_ref[...], preferred_element_type=jnp.float32)
        + b_ref[...], 0.0)
    g = dis * jnp.dot(a, w_h0_ref[...], preferred_element_type=jnp.float32)
    o0_ref[...] = g[:, :HALF]
    o1_ref[...] = g[:, HALF:]


_tc_l1 = pl.pallas_call(
    _tc_l1_body,
    grid=(GRID,),
    in_specs=_pad_specs(W1) + [
        _node_spec(W1), _full_spec((W1, HID)), _full_spec((HID, HID)),
        _full_spec((1, HID)), _node_spec(1), _node_spec(1),
    ],
    out_specs=[_node_spec(HALF), _node_spec(HALF)],
    out_shape=[jax.ShapeDtypeStruct((N, HALF), jnp.float32)] * 2,
)


def _tc_mid_body(s0_ref, s1_ref, g0_ref, g1_ref, w_ref, b_ref,
                 d0_ref, d1_ref, o0_ref, o1_ref):
    dis = _dis(d0_ref[...], d1_ref[...])
    ssum0 = s0_ref[0] + g0_ref[...]
    ssum1 = s1_ref[0] + g1_ref[...]
    pre = dis * jnp.concatenate([ssum0, ssum1], axis=1) + b_ref[...]
    a = jnp.maximum(pre, 0.0)
    g = dis * jnp.dot(a, w_ref[...], preferred_element_type=jnp.float32)
    o0_ref[...] = g[:, :HALF]
    o1_ref[...] = g[:, HALF:]


_tc_mid = pl.pallas_call(
    _tc_mid_body,
    grid=(GRID,),
    in_specs=_pad_specs(HALF) + [
        _node_spec(HALF), _node_spec(HALF), _full_spec((HID, HID)),
        _full_spec((1, HID)), _node_spec(1), _node_spec(1),
    ],
    out_specs=[_node_spec(HALF), _node_spec(HALF)],
    out_shape=[jax.ShapeDtypeStruct((N, HALF), jnp.float32)] * 2,
)


def _tc_l4_body(s0_ref, s1_ref, g0_ref, g1_ref, w_ref, b_ref,
                d0_ref, d1_ref, o_ref):
    dis = _dis(d0_ref[...], d1_ref[...])
    ssum0 = s0_ref[0] + g0_ref[...]
    ssum1 = s1_ref[0] + g1_ref[...]
    pre = dis * jnp.concatenate([ssum0, ssum1], axis=1) + b_ref[...]
    a = jnp.maximum(pre, 0.0)
    o_ref[...] = dis * jnp.dot(a, w_ref[...], preferred_element_type=jnp.float32)


_tc_l4 = pl.pallas_call(
    _tc_l4_body,
    grid=(GRID,),
    in_specs=_pad_specs(HALF) + [
        _node_spec(HALF), _node_spec(HALF), _full_spec((HID, 1)),
        _full_spec((1, HID)), _node_spec(1), _node_spec(1),
    ],
    out_specs=_node_spec(1),
    out_shape=jax.ShapeDtypeStruct((N, 1), jnp.float32),
)


def _tc_out_body(s0_ref, s1_ref, g4_ref, b_ref, d0_ref, d1_ref, o_ref):
    dis = _dis(d0_ref[...], d1_ref[...])
    o_ref[...] = dis * (s0_ref[...] + s1_ref[...] + g4_ref[...]) + b_ref[...]


_tc_out = pl.pallas_call(
    _tc_out_body,
    grid=(GRID,),
    in_specs=[
        _node_spec(1), _node_spec(1), _node_spec(1), _full_spec((1, 1)),
        _node_spec(1), _node_spec(1),
    ],
    out_specs=_node_spec(1),
    out_shape=jax.ShapeDtypeStruct((N, 1), jnp.float32),
)


def kernel(x, edge_index, W_in, b_in, W_h0, b_h0, W_h1, b_h1, W_out, b_out):
    src = edge_index[0]
    dst = edge_index[1]

    ones_deg = jnp.ones((EPW,), jnp.float32)
    deg = _sc_degree(dst, ones_deg)
    d0 = deg[:N].reshape(N, 1)
    d1 = deg[NPAD:NPAD + N].reshape(N, 1)

    b_in2 = b_in.reshape(1, HID)
    b_h02 = b_h0.reshape(1, HID)
    b_h12 = b_h1.reshape(1, HID)

    zeros16 = jnp.zeros((RPT, W1), jnp.float32)
    zeros1 = jnp.zeros((RPT,), jnp.float32)
    W_inp = jnp.concatenate(
        [W_in, jnp.zeros((W1 - 4, HID), jnp.float32)], axis=0)

    gx = _tc_gx(x, d0, d1)
    Sx = _sc_agg16(src, dst, gx, zeros16)
    g0, g1 = _tc_l1(Sx, Sx, gx, W_inp, W_h0, b_in2, d0, d1)
    S2 = _sc_agg_wide(src, dst, g0, g1)
    g0, g1 = _tc_mid(S2, S2, g0, g1, W_h1, b_h02, d0, d1)
    S3 = _sc_agg_wide(src, dst, g0, g1)
    g4 = _tc_l4(S3, S3, g0, g1, W_out, b_h12, d0, d1)
    S4 = _sc_agg1(src, dst, g4.reshape(N), zeros1)
    s40 = S4[:N].reshape(N, 1)
    s41 = S4[NPAD:NPAD + N].reshape(N, 1)
    return _tc_out(s40, s41, g4, b_out.reshape(1, 1), d0, d1)


# speedup vs baseline: 33.7262x; 1.1614x over previous
import functools

import jax
import jax.numpy as jnp
from jax import lax
from jax.experimental import pallas as pl
from jax.experimental.pallas import tpu as pltpu
from jax.experimental.pallas import tpu_sc as plsc

N = 50000
E = 800000
HID = 64
HALF = 32

NC = 2
NS = 16

RPT = 3200
NPAD = RPT * NS
CH = 200
NCH = RPT // CH

EPT = E // NS
EB = 200
NB = EPT // EB
NBUF = 4

EPW = E // (NC * NS)
W1 = 16
NBUFN = 5

_mesh = plsc.VectorSubcoreMesh(
    core_axis_name="c", subcore_axis_name="s", num_cores=NC, num_subcores=NS
)
_sc_params = pltpu.CompilerParams(use_tc_tiling_on_sc=False)


def _zero_fill(ref, nrows, width):
    per_row = width // 16
    if per_row == 0:
        def body(i, carry):
            ref[pl.ds(i * 16, 16)] = jnp.zeros((16,), jnp.float32)
            return carry
        lax.fori_loop(0, nrows * width // 16, body, 0)
    else:
        def body(i, carry):
            for k in range(per_row):
                ref[i, pl.ds(k * 16, 16)] = jnp.zeros((16,), jnp.float32)
            return carry
        lax.fori_loop(0, nrows, body, 0)


@functools.partial(
    pl.kernel,
    out_type=jax.ShapeDtypeStruct((NC * NPAD,), jnp.float32),
    mesh=_mesh,
    scratch_types=[
        pltpu.VMEM((EPW,), jnp.int32),
        pltpu.VMEM((EPW,), jnp.float32),
        pltpu.VMEM_SHARED((NPAD,), jnp.float32),
    ],
    compiler_params=_sc_params,
)
def _sc_degree(dst_hbm, ones_hbm, zeros_hbm, deg_hbm, dst_v, ones_v, acc_sh):
    c = lax.axis_index("c")
    s = lax.axis_index("s")
    row = s * RPT
    pltpu.sync_copy(zeros_hbm, acc_sh.at[pl.ds(row, RPT)])
    plsc.subcore_barrier()

    wid = c * NS + s
    pltpu.sync_copy(dst_hbm.at[pl.ds(wid * EPW, EPW)], dst_v)
    pltpu.sync_copy(ones_hbm, ones_v)
    pltpu.sync_copy(ones_v, acc_sh.at[dst_v], add=True)
    plsc.subcore_barrier()

    pltpu.sync_copy(acc_sh.at[pl.ds(row, RPT)],
                    deg_hbm.at[pl.ds(c * NPAD + row, RPT)])


@functools.partial(
    pl.kernel,
    out_type=jax.ShapeDtypeStruct((NC, NPAD, HALF), jnp.float32),
    mesh=_mesh,
    scratch_types=[
        [pltpu.VMEM((EB,), jnp.int32)] * NBUF,
        [pltpu.VMEM((EB,), jnp.int32)] * NBUF,
        [pltpu.VMEM((EB, HALF), jnp.float32)] * NBUF,
        pltpu.VMEM_SHARED((NPAD, HALF), jnp.float32),
        [pltpu.SemaphoreType.DMA] * NBUF,
        [pltpu.SemaphoreType.DMA] * NBUF,
        [pltpu.SemaphoreType.DMA] * NBUF,
    ],
    compiler_params=_sc_params,
)
def _sc_agg_wide(src_hbm, dst_hbm, g0_hbm, g1_hbm, zeros_hbm, out_hbm,
                 src_v, dst_v, rows_v, acc_sh, isem, gsem, ssem):
    c = lax.axis_index("c")
    s = lax.axis_index("s")
    row = s * RPT
    pltpu.sync_copy(zeros_hbm, acc_sh.at[pl.ds(row, RPT)])
    plsc.subcore_barrier()

    def outer(o, carry):
        for b in range(NBUF):
            base = s * EPT + (o * NBUF + b) * EB

            @pl.when(o > 0)
            def _():
                pltpu.make_async_copy(
                    rows_v[b], acc_sh.at[dst_v[b]], ssem[b]).wait()

            pltpu.async_copy(src_hbm.at[pl.ds(base, EB)], src_v[b], isem[b])
            pltpu.async_copy(dst_hbm.at[pl.ds(base, EB)], dst_v[b], isem[b])
        for b in range(NBUF):
            base = s * EPT + (o * NBUF + b) * EB
            pltpu.make_async_copy(
                src_hbm.at[pl.ds(base, EB)], src_v[b], isem[b]).wait()
            pltpu.make_async_copy(
                dst_hbm.at[pl.ds(base, EB)], dst_v[b], isem[b]).wait()

            @pl.when(c == 0)
            def _():
                pltpu.async_copy(g0_hbm.at[src_v[b]], rows_v[b], gsem[b])

            @pl.when(c == 1)
            def _():
                pltpu.async_copy(g1_hbm.at[src_v[b]], rows_v[b], gsem[b])
        for b in range(NBUF):
            pltpu.make_async_copy(
                g0_hbm.at[src_v[b]], rows_v[b], gsem[b]).wait()
            pltpu.async_copy(rows_v[b], acc_sh.at[dst_v[b]], ssem[b], add=True)
        return carry

    lax.fori_loop(0, NB // NBUF, outer, 0)
    for b in range(NBUF):
        pltpu.make_async_copy(rows_v[b], acc_sh.at[dst_v[b]], ssem[b]).wait()
    plsc.subcore_barrier()
    pltpu.sync_copy(acc_sh.at[pl.ds(row, RPT)], out_hbm.at[c, pl.ds(row, RPT)])


def _make_sc_agg_narrow(width, ebn):
    out3 = width > 1
    nbn = EPW // ebn
    out_shape = (NC, NPAD, width) if out3 else (NC * NPAD,)
    rows_shape = (ebn, width) if out3 else (ebn,)
    acc_shape = (NPAD, width) if out3 else (NPAD,)
    stg_shape = (RPT, width) if out3 else (RPT,)

    @functools.partial(
        pl.kernel,
        out_type=jax.ShapeDtypeStruct(out_shape, jnp.float32),
        mesh=_mesh,
        scratch_types=[
            [pltpu.VMEM((ebn,), jnp.int32)] * NBUFN,
            [pltpu.VMEM((ebn,), jnp.int32)] * NBUFN,
            [pltpu.VMEM(rows_shape, jnp.float32)] * NBUFN,
            pltpu.VMEM_SHARED(acc_shape, jnp.float32),
            [pltpu.SemaphoreType.DMA] * NBUFN,
            [pltpu.SemaphoreType.DMA] * NBUFN,
            [pltpu.SemaphoreType.DMA] * NBUFN,
        ],
        compiler_params=_sc_params,
    )
    def agg(src_hbm, dst_hbm, g_hbm, zeros_hbm, out_hbm,
            src_v, dst_v, rows_v, acc_sh, isem, gsem, ssem):
        c = lax.axis_index("c")
        s = lax.axis_index("s")
        row = s * RPT
        pltpu.sync_copy(zeros_hbm, acc_sh.at[pl.ds(row, RPT)])
        plsc.subcore_barrier()

        def outer(o, carry):
            for b in range(NBUFN):
                base = (c * NS + s) * EPW + (o * NBUFN + b) * ebn

                @pl.when(o > 0)
                def _():
                    pltpu.make_async_copy(
                        rows_v[b], acc_sh.at[dst_v[b]], ssem[b]).wait()

                pltpu.async_copy(src_hbm.at[pl.ds(base, ebn)], src_v[b], isem[b])
                pltpu.async_copy(dst_hbm.at[pl.ds(base, ebn)], dst_v[b], isem[b])
            for b in range(NBUFN):
                base = (c * NS + s) * EPW + (o * NBUFN + b) * ebn
                pltpu.make_async_copy(
                    src_hbm.at[pl.ds(base, ebn)], src_v[b], isem[b]).wait()
                pltpu.make_async_copy(
                    dst_hbm.at[pl.ds(base, ebn)], dst_v[b], isem[b]).wait()
                pltpu.async_copy(g_hbm.at[src_v[b]], rows_v[b], gsem[b])
            for b in range(NBUFN):
                pltpu.make_async_copy(
                    g_hbm.at[src_v[b]], rows_v[b], gsem[b]).wait()
                pltpu.async_copy(rows_v[b], acc_sh.at[dst_v[b]], ssem[b],
                                 add=True)
            return carry

        lax.fori_loop(0, nbn // NBUFN, outer, 0)
        for b in range(NBUFN):
            pltpu.make_async_copy(rows_v[b], acc_sh.at[dst_v[b]], ssem[b]).wait()
        plsc.subcore_barrier()

        if out3:
            pltpu.sync_copy(acc_sh.at[pl.ds(row, RPT)],
                            out_hbm.at[c, pl.ds(row, RPT)])
        else:
            pltpu.sync_copy(acc_sh.at[pl.ds(row, RPT)],
                            out_hbm.at[pl.ds(c * NPAD + row, RPT)])

    return agg


_sc_agg16 = _make_sc_agg_narrow(W1, 200)
_sc_agg1 = _make_sc_agg_narrow(1, 1000)


BLK = 2000
GRID = N // BLK


def _dis(d0_blk, d1_blk):
    return lax.rsqrt(d0_blk + d1_blk + 1.0)


def _node_spec(width):
    return pl.BlockSpec((BLK, width), lambda i: (i, 0))


def _pad_specs(width):
    return [
        pl.BlockSpec((1, BLK, width), lambda i: (0, i, 0)),
        pl.BlockSpec((1, BLK, width), lambda i: (1, i, 0)),
    ]


def _full_spec(shape):
    return pl.BlockSpec(shape, lambda i: tuple(0 for _ in shape))


def _tc_gx_body(x_ref, d0_ref, d1_ref, gx_ref):
    gx = _dis(d0_ref[...], d1_ref[...]) * x_ref[...]
    gx_ref[...] = jnp.concatenate(
        [gx, jnp.zeros((BLK, W1 - 4), jnp.float32)], axis=1)


_tc_gx = pl.pallas_call(
    _tc_gx_body,
    grid=(GRID,),
    in_specs=[_node_spec(4), _node_spec(1), _node_spec(1)],
    out_specs=_node_spec(W1),
    out_shape=jax.ShapeDtypeStruct((N, W1), jnp.float32),
)


def _tc_l1_body(sx0_ref, sx1_ref, gx_ref, w_in_ref, w_h0_ref, b_ref,
                d0_ref, d1_ref, o0_ref, o1_ref):
    dis = _dis(d0_ref[...], d1_ref[...])
    pre = dis * (sx0_ref[0] + sx1_ref[0] + gx_ref[...])
    a = jnp.maximum(
        jnp.dot(pre, w_in_ref[...], preferred_element_type=jnp.float32)
        + b_ref[...], 0.0)
    g = dis * jnp.dot(a, w_h0_ref[...], preferred_element_type=jnp.float32)
    o0_ref[...] = g[:, :HALF]
    o1_ref[...] = g[:, HALF:]


_tc_l1 = pl.pallas_call(
    _tc_l1_body,
    grid=(GRID,),
    in_specs=_pad_specs(W1) + [
        _node_spec(W1), _full_spec((W1, HID)), _full_spec((HID, HID)),
        _full_spec((1, HID)), _node_spec(1), _node_spec(1),
    ],
    out_specs=[_node_spec(HALF), _node_spec(HALF)],
    out_shape=[jax.ShapeDtypeStruct((N, HALF), jnp.float32)] * 2,
)


def _tc_mid_body(s0_ref, s1_ref, g0_ref, g1_ref, w_ref, b_ref,
                 d0_ref, d1_ref, o0_ref, o1_ref):
    dis = _dis(d0_ref[...], d1_ref[...])
    ssum0 = s0_ref[0] + g0_ref[...]
    ssum1 = s1_ref[0] + g1_ref[...]
    pre = dis * jnp.concatenate([ssum0, ssum1], axis=1) + b_ref[...]
    a = jnp.maximum(pre, 0.0)
    g = dis * jnp.dot(a, w_ref[...], preferred_element_type=jnp.float32)
    o0_ref[...] = g[:, :HALF]
    o1_ref[...] = g[:, HALF:]


_tc_mid = pl.pallas_call(
    _tc_mid_body,
    grid=(GRID,),
    in_specs=_pad_specs(HALF) + [
        _node_spec(HALF), _node_spec(HALF), _full_spec((HID, HID)),
        _full_spec((1, HID)), _node_spec(1), _node_spec(1),
    ],
    out_specs=[_node_spec(HALF), _node_spec(HALF)],
    out_shape=[jax.ShapeDtypeStruct((N, HALF), jnp.float32)] * 2,
)


def _tc_l4_body(s0_ref, s1_ref, g0_ref, g1_ref, w_ref, b_ref,
                d0_ref, d1_ref, o_ref):
    dis = _dis(d0_ref[...], d1_ref[...])
    ssum0 = s0_ref[0] + g0_ref[...]
    ssum1 = s1_ref[0] + g1_ref[...]
    pre = dis * jnp.concatenate([ssum0, ssum1], axis=1) + b_ref[...]
    a = jnp.maximum(pre, 0.0)
    o_ref[...] = dis * jnp.dot(a, w_ref[...], preferred_element_type=jnp.float32)


_tc_l4 = pl.pallas_call(
    _tc_l4_body,
    grid=(GRID,),
    in_specs=_pad_specs(HALF) + [
        _node_spec(HALF), _node_spec(HALF), _full_spec((HID, 1)),
        _full_spec((1, HID)), _node_spec(1), _node_spec(1),
    ],
    out_specs=_node_spec(1),
    out_shape=jax.ShapeDtypeStruct((N, 1), jnp.float32),
)


def _tc_out_body(s0_ref, s1_ref, g4_ref, b_ref, d0_ref, d1_ref, o_ref):
    dis = _dis(d0_ref[...], d1_ref[...])
    o_ref[...] = dis * (s0_ref[...] + s1_ref[...] + g4_ref[...]) + b_ref[...]


_tc_out = pl.pallas_call(
    _tc_out_body,
    grid=(GRID,),
    in_specs=[
        _node_spec(1), _node_spec(1), _node_spec(1), _full_spec((1, 1)),
        _node_spec(1), _node_spec(1),
    ],
    out_specs=_node_spec(1),
    out_shape=jax.ShapeDtypeStruct((N, 1), jnp.float32),
)


def kernel(x, edge_index, W_in, b_in, W_h0, b_h0, W_h1, b_h1, W_out, b_out):
    src = edge_index[0]
    dst = edge_index[1]

    ones_deg = jnp.ones((EPW,), jnp.float32)
    zeros1 = jnp.zeros((RPT,), jnp.float32)
    deg = _sc_degree(dst, ones_deg, zeros1)
    d0 = deg[:N].reshape(N, 1)
    d1 = deg[NPAD:NPAD + N].reshape(N, 1)

    b_in2 = b_in.reshape(1, HID)
    b_h02 = b_h0.reshape(1, HID)
    b_h12 = b_h1.reshape(1, HID)

    zeros16 = jnp.zeros((RPT, W1), jnp.float32)
    zeros32 = jnp.zeros((RPT, HALF), jnp.float32)
    W_inp = jnp.concatenate(
        [W_in, jnp.zeros((W1 - 4, HID), jnp.float32)], axis=0)

    gx = _tc_gx(x, d0, d1)
    Sx = _sc_agg16(src, dst, gx, zeros16)
    g0, g1 = _tc_l1(Sx, Sx, gx, W_inp, W_h0, b_in2, d0, d1)
    S2 = _sc_agg_wide(src, dst, g0, g1, zeros32)
    g0, g1 = _tc_mid(S2, S2, g0, g1, W_h1, b_h02, d0, d1)
    S3 = _sc_agg_wide(src, dst, g0, g1, zeros32)
    g4 = _tc_l4(S3, S3, g0, g1, W_out, b_h12, d0, d1)
    S4 = _sc_agg1(src, dst, g4.reshape(N), zeros1)
    s40 = S4[:N].reshape(N, 1)
    s41 = S4[NPAD:NPAD + N].reshape(N, 1)
    return _tc_out(s40, s41, g4, b_out.reshape(1, 1), d0, d1)
